# baseline (speedup 1.0000x reference)
import functools

import jax
import jax.numpy as jnp
from jax import lax
from jax.experimental import pallas as pl
from jax.experimental.pallas import tpu as pltpu
from jax.experimental.pallas import tpu_sc as plsc

B = 16384
D = 16
L = 16
NC = 2
NS = 16
NW = NC * NS
BPW = B // NW
NCH = BPW // 128

_LOG1P_COEF = (
    9.09903358e-08, 9.99991449e-01, -4.99801099e-01, 3.31333659e-01,
    -2.39189722e-01, 1.64781887e-01, -9.23123095e-02, 3.44179115e-02,
    -6.07475245e-03,
)


def _softplus(x):
    t = jnp.exp(-jnp.abs(x))
    p = jnp.full((L,), _LOG1P_COEF[-1], jnp.float32)
    for c in _LOG1P_COEF[-2::-1]:
        p = p * t + c
    return jnp.maximum(x, 0.0) + p


@functools.partial(
    pl.kernel,
    out_type=jax.ShapeDtypeStruct((B,), jnp.float32),
    mesh=plsc.VectorSubcoreMesh(core_axis_name="c", subcore_axis_name="s"),
    compiler_params=pltpu.CompilerParams(
        needs_layout_passes=False, use_tc_tiling_on_sc=False),
    scratch_types=[
        pltpu.VMEM((NCH, 128), jnp.int32),
        pltpu.VMEM((NCH, 128), jnp.int32),
        pltpu.VMEM((BPW, D), jnp.float32),
        pltpu.VMEM((BPW, D), jnp.float32),
        pltpu.VMEM((BPW,), jnp.float32),
        pltpu.VMEM((BPW,), jnp.float32),
        pltpu.SemaphoreType.DMA,
    ],
)
def _mirt_sc(user_hbm, item_hbm, theta_hbm, a_hbm, b_hbm, out_hbm,
             uidx_v, iidx_v, th_v, a_v, b_v, out_v, sem):
    wid = lax.axis_index("s") * NC + lax.axis_index("c")

    pltpu.sync_copy(user_hbm.at[pl.ds(wid * NCH, NCH)], uidx_v)
    pltpu.sync_copy(item_hbm.at[pl.ds(wid * NCH, NCH)], iidx_v)

    copies = []
    for j in range(NCH):
        sl = pl.ds(j * 128, 128)
        copies.append(pltpu.async_copy(theta_hbm.at[uidx_v.at[j]], th_v.at[sl], sem))
        copies.append(pltpu.async_copy(a_hbm.at[iidx_v.at[j]], a_v.at[sl], sem))
        copies.append(pltpu.async_copy(b_hbm.at[iidx_v.at[j]], b_v.at[sl], sem))
    for c in copies:
        c.wait()

    def group_body(g, carry):
        rows = lax.iota(jnp.int32, L) + g * L

        def k_body(k, acc):
            cols = jnp.full((L,), k, jnp.int32)
            va = plsc.load_gather(a_v, [rows, cols])
            vt = plsc.load_gather(th_v, [rows, cols])
            return acc + _softplus(va) * vt

        acc = lax.fori_loop(0, D, k_body, jnp.zeros((L,), jnp.float32))
        vb = plsc.load_gather(b_v, [rows])
        res = 1.0 / (1.0 + jnp.exp(vb - acc))
        plsc.store_scatter(out_v, [rows], res)
        return carry

    lax.fori_loop(0, BPW // L, group_body, 0)
    pltpu.sync_copy(out_v, out_hbm.at[pl.ds(wid * BPW, BPW)])


def kernel(user, item, theta_table, a_table, b_table):
    u2 = user.astype(jnp.int32).reshape(NW * NCH, 128)
    i2 = item.astype(jnp.int32).reshape(NW * NCH, 128)
    b1 = b_table.reshape((b_table.shape[0],))
    return _mirt_sc(u2, i2, theta_table, a_table, b1)
